# Initial kernel scaffold; baseline (speedup 1.0000x reference)
import jax
import jax.numpy as jnp
from jax.experimental import pallas as pl


def kernel(vis_feat, ir_feat, params):
    raise NotImplementedError("write your pallas kernel here")


# speedup vs baseline: 1.1501x; 1.1501x over previous
import functools
import jax
import jax.numpy as jnp
from jax.experimental import pallas as pl

B, C, H, W = 2, 96, 224, 224
P = 4
HP = H // P
N = HP * HP
NPAD = 3200
TD = C * P * P
EMBED = 128
HEADS = 4
DH = EMBED // HEADS
K = int(N * 0.25)
TBLK = 640
NT = NPAD // TBLK
KCH = 640
RBH = 32
NRB = H // RBH
NEG = -1e30

_f32 = jnp.float32
_HI = jax.lax.Precision.HIGHEST


def _fft_body(x_ref, fr_ref, fi_ref, amp_ref, ph_ref, mean_ref):
    x = x_ref[0]
    fr = fr_ref[...]
    fi = fi_ref[...]
    dot = functools.partial(jnp.dot, preferred_element_type=_f32, precision=_HI)
    g1 = dot(fr, x)
    g2 = dot(fi, x)
    xr = dot(g1, fr) - dot(g2, fi)
    xi = dot(g1, fi) + dot(g2, fr)
    amp_ref[0] = jnp.sqrt(xr * xr + xi * xi)
    ph_ref[0] = jnp.arctan2(xi, xr)
    mean_ref[0] = (jnp.sum(x) / (H * W)).reshape(1, 1)


def _fft(x_all, fr, fi):
    nimg = x_all.shape[0]
    return pl.pallas_call(
        _fft_body,
        grid=(nimg,),
        in_specs=[
            pl.BlockSpec((1, H, W), lambda i: (i, 0, 0)),
            pl.BlockSpec((H, W), lambda i: (0, 0)),
            pl.BlockSpec((H, W), lambda i: (0, 0)),
        ],
        out_specs=[
            pl.BlockSpec((1, H, W), lambda i: (i, 0, 0)),
            pl.BlockSpec((1, H, W), lambda i: (i, 0, 0)),
            pl.BlockSpec((1, 1, 1), lambda i: (i, 0, 0)),
        ],
        out_shape=[
            jax.ShapeDtypeStruct((nimg, H, W), _f32),
            jax.ShapeDtypeStruct((nimg, H, W), _f32),
            jax.ShapeDtypeStruct((nimg, 1, 1), _f32),
        ],
    )(x_all, fr, fi)


def _proj_body(vt_ref, it_ref, w1v_ref, w1i_ref, cb_ref, ib_ref, w2_ref,
               b2_ref, wq_ref, bq_ref, wk_ref, bk_ref, wv_ref, bv_ref,
               q_ref, k_ref, v_ref, s_ref):
    vt = vt_ref[0]
    it = it_ref[0]
    dot = functools.partial(jnp.dot, preferred_element_type=_f32, precision=_HI)
    h = dot(vt, w1v_ref[...]) + dot(it, w1i_ref[...]) + cb_ref[...] + ib_ref[0]
    h = jax.nn.relu(h)
    s_ref[0] = dot(h, w2_ref[...]) + b2_ref[...]
    q_ref[0] = dot(vt, wq_ref[...]) + bq_ref[...]
    k_ref[0] = dot(it, wk_ref[...]) + bk_ref[...]
    v_ref[0] = dot(it, wv_ref[...]) + bv_ref[...]


def _proj(vt, it, w1v, w1i, cb, ib, w2, b2, wq, bq, wk, bk, wv, bv):
    full = pl.BlockSpec
    wspec = full((TD, EMBED), lambda b, t: (0, 0))
    bspec = full((1, EMBED), lambda b, t: (0, 0))
    qkv = full((1, TBLK, EMBED), lambda b, t: (b, t, 0))
    return pl.pallas_call(
        _proj_body,
        grid=(B, NT),
        in_specs=[
            full((1, TBLK, TD), lambda b, t: (b, t, 0)),
            full((1, TBLK, TD), lambda b, t: (b, t, 0)),
            wspec, wspec,
            full((TBLK, EMBED), lambda b, t: (t, 0)),
            full((1, 1, EMBED), lambda b, t: (b, 0, 0)),
            full((EMBED, 1), lambda b, t: (0, 0)),
            full((1, 1), lambda b, t: (0, 0)),
            wspec, bspec, wspec, bspec, wspec, bspec,
        ],
        out_specs=[
            qkv, qkv, qkv,
            full((1, TBLK, 1), lambda b, t: (b, t, 0)),
        ],
        out_shape=[
            jax.ShapeDtypeStruct((B, NPAD, EMBED), _f32),
            jax.ShapeDtypeStruct((B, NPAD, EMBED), _f32),
            jax.ShapeDtypeStruct((B, NPAD, EMBED), _f32),
            jax.ShapeDtypeStruct((B, NPAD, 1), _f32),
        ],
    )(vt, it, w1v, w1i, cb, ib, w2, b2, wq, bq, wk, bk, wv, bv)


def _topk_body(s_ref, mrow_ref, mcol_ref):
    def ibody(ic, carry):
        si_row = s_ref[0, :, pl.ds(ic * 128, 128)]
        si_col = jnp.transpose(si_row, (1, 0))
        ii = jax.lax.broadcasted_iota(jnp.int32, (128, 1), 0) + ic * 128

        def jbody(jc, acc):
            sj = s_ref[0, :, pl.ds(jc * 640, 640)]
            jj = jax.lax.broadcasted_iota(jnp.int32, (1, 640), 1) + jc * 640
            gt = (sj > si_col) | ((sj == si_col) & (jj < ii))
            return acc + jnp.sum(gt.astype(_f32), axis=1, keepdims=True)

        rank = jax.lax.fori_loop(0, NPAD // 640, jbody,
                                 jnp.zeros((128, 1), _f32))
        mask = (rank < float(K)).astype(_f32)
        mcol_ref[0, pl.ds(ic * 128, 128), :] = mask
        mrow_ref[0, :, pl.ds(ic * 128, 128)] = jnp.transpose(mask, (1, 0))
        return carry

    jax.lax.fori_loop(0, NPAD // 128, ibody, 0)


def _topk(srow):
    return pl.pallas_call(
        _topk_body,
        grid=(B,),
        in_specs=[pl.BlockSpec((1, 1, NPAD), lambda b: (b, 0, 0))],
        out_specs=[
            pl.BlockSpec((1, 1, NPAD), lambda b: (b, 0, 0)),
            pl.BlockSpec((1, NPAD, 1), lambda b: (b, 0, 0)),
        ],
        out_shape=[
            jax.ShapeDtypeStruct((B, 1, NPAD), _f32),
            jax.ShapeDtypeStruct((B, NPAD, 1), _f32),
        ],
    )(srow)


def _attn_body(q_ref, k_ref, v_ref, mrow_ref, mcol_ref, vt_ref, it_ref,
               wo_ref, bo_ref, o_ref):
    dotg = jax.lax.dot_general
    q = q_ref[0] * (1.0 / jnp.sqrt(float(DH)))
    outs = []
    for h in range(HEADS):
        qh = q[:, h * DH:(h + 1) * DH]
        m0 = jnp.full((TBLK, 1), NEG, _f32)
        l0 = jnp.zeros((TBLK, 1), _f32)
        a0 = jnp.zeros((TBLK, DH), _f32)

        def kbody(jc, carry, qh=qh, h=h):
            m, l, acc = carry
            kc = k_ref[0, pl.ds(jc * KCH, KCH), h * DH:(h + 1) * DH]
            vc = v_ref[0, pl.ds(jc * KCH, KCH), h * DH:(h + 1) * DH]
            mb = mrow_ref[0, :, pl.ds(jc * KCH, KCH)]
            s = dotg(qh, kc, (((1,), (1,)), ((), ())),
                     preferred_element_type=_f32, precision=_HI)
            s = s + (mb - 1.0) * (-NEG)
            mnew = jnp.maximum(m, jnp.max(s, axis=1, keepdims=True))
            p = jnp.exp(s - mnew)
            corr = jnp.exp(m - mnew)
            lnew = l * corr + jnp.sum(p, axis=1, keepdims=True)
            accnew = acc * corr + jnp.dot(p, vc, preferred_element_type=_f32, precision=_HI)
            return mnew, lnew, accnew

        m, l, acc = jax.lax.fori_loop(0, NPAD // KCH, kbody, (m0, l0, a0))
        outs.append(acc / l)
    o = jnp.concatenate(outs, axis=1)
    attn = jnp.dot(o, wo_ref[...], preferred_element_type=_f32, precision=_HI) + bo_ref[...]
    base = 0.5 * (vt_ref[0] + it_ref[0])
    o_ref[0] = base + mcol_ref[0] * attn


def _attn(q, k, v, mrow, mcol, vt, it, wo, bo):
    full = pl.BlockSpec
    return pl.pallas_call(
        _attn_body,
        grid=(B, NT),
        in_specs=[
            full((1, TBLK, EMBED), lambda b, t: (b, t, 0)),
            full((1, NPAD, EMBED), lambda b, t: (b, 0, 0)),
            full((1, NPAD, EMBED), lambda b, t: (b, 0, 0)),
            full((1, 1, NPAD), lambda b, t: (b, 0, 0)),
            full((1, TBLK, 1), lambda b, t: (b, t, 0)),
            full((1, TBLK, TD), lambda b, t: (b, t, 0)),
            full((1, TBLK, TD), lambda b, t: (b, t, 0)),
            full((EMBED, TD), lambda b, t: (0, 0)),
            full((1, TD), lambda b, t: (0, 0)),
        ],
        out_specs=full((1, TBLK, TD), lambda b, t: (b, t, 0)),
        out_shape=jax.ShapeDtypeStruct((B, NPAD, TD), _f32),
    )(q, k, v, mrow, mcol, vt, it, wo, bo)


def _ifft_body(fa_ref, fp_ref, fr_ref, fi_ref, o_ref):
    fa = fa_ref[0]
    fp = fp_ref[0]
    fr = fr_ref[...]
    fi = fi_ref[...]
    dot = functools.partial(jnp.dot, preferred_element_type=_f32, precision=_HI)
    cr = fa * jnp.cos(fp)
    ci = fa * jnp.sin(fp)
    mr = dot(fr, cr) + dot(fi, ci)
    mi = dot(fr, ci) - dot(fi, cr)
    o_ref[0] = dot(mr, fr) + dot(mi, fi)


def _ifft(fa, fp, fr, fi):
    nimg = fa.shape[0]
    return pl.pallas_call(
        _ifft_body,
        grid=(nimg,),
        in_specs=[
            pl.BlockSpec((1, H, W), lambda i: (i, 0, 0)),
            pl.BlockSpec((1, H, W), lambda i: (i, 0, 0)),
            pl.BlockSpec((H, W), lambda i: (0, 0)),
            pl.BlockSpec((H, W), lambda i: (0, 0)),
        ],
        out_specs=pl.BlockSpec((1, H, W), lambda i: (i, 0, 0)),
        out_shape=jax.ShapeDtypeStruct((nimg, H, W), _f32),
    )(fa, fp, fr, fi)


def _conv_body(xp_ref, xc_ref, xn_ref, w_ref, b_ref, *rest):
    residual = len(rest) == 3
    if residual:
        vis_ref, ir_ref, o_ref = rest
    else:
        (o_ref,) = rest
    xall = jnp.concatenate([xp_ref[0], xc_ref[0], xn_ref[0]], axis=1)
    r = pl.program_id(1)
    acc = jnp.zeros((C, RBH * W), _f32)
    for ky in range(3):
        win = xall[:, RBH + ky - 1:2 * RBH + ky - 1, :]
        gi = (RBH * r + (ky - 1)
              + jax.lax.broadcasted_iota(jnp.int32, (1, RBH, 1), 1))
        win = win * ((gi >= 0) & (gi < H)).astype(_f32)
        for kx in range(3):
            if kx == 0:
                wc = jnp.concatenate(
                    [jnp.zeros((C, RBH, 1), _f32), win[:, :, :W - 1]], axis=2)
            elif kx == 1:
                wc = win
            else:
                wc = jnp.concatenate(
                    [win[:, :, 1:], jnp.zeros((C, RBH, 1), _f32)], axis=2)
            wk = w_ref[ky * 3 + kx]
            acc = acc + jax.lax.dot_general(
                wk, wc.reshape(C, RBH * W), (((1,), (0,)), ((), ())),
                preferred_element_type=_f32, precision=_HI)
    y = acc.reshape(C, RBH, W) + b_ref[...].reshape(C, 1, 1)
    if residual:
        y = y + 0.5 * (vis_ref[0] + ir_ref[0])
    else:
        y = jax.nn.relu(y)
    o_ref[0] = y


def _conv(x, w9, bias, vis=None, ir=None):
    full = pl.BlockSpec
    xspec = lambda fn: full((1, C, RBH, W), fn)
    clamp = lambda v: jnp.clip(v, 0, NRB - 1)
    in_specs = [
        xspec(lambda b, r: (b, 0, clamp(r - 1), 0)),
        xspec(lambda b, r: (b, 0, r, 0)),
        xspec(lambda b, r: (b, 0, clamp(r + 1), 0)),
        full((9, C, C), lambda b, r: (0, 0, 0)),
        full((C, 1), lambda b, r: (0, 0)),
    ]
    args = [x, x, x, w9, bias]
    if vis is not None:
        in_specs += [xspec(lambda b, r: (b, 0, r, 0)),
                     xspec(lambda b, r: (b, 0, r, 0))]
        args += [vis, ir]
    return pl.pallas_call(
        _conv_body,
        grid=(B, NRB),
        in_specs=in_specs,
        out_specs=xspec(lambda b, r: (b, 0, r, 0)),
        out_shape=jax.ShapeDtypeStruct((B, C, H, W), _f32),
    )(*args)


def _patchify(x):
    t = x.reshape(B, C, HP, P, HP, P)
    return t.transpose(0, 2, 4, 1, 3, 5).reshape(B, N, TD)


def _unpatchify(t):
    t = t.reshape(B, HP, HP, C, P, P).transpose(0, 3, 1, 4, 2, 5)
    return t.reshape(B, C, H, W)


def _pad_tok(t):
    return jnp.pad(t, ((0, 0), (0, NPAD - N), (0, 0)))


def _branch(va, ia, intent, coords, sp, ip_):
    vt = _pad_tok(_patchify(va))
    it = _pad_tok(_patchify(ia))
    w1 = sp['w1']
    cb = coords @ w1[2 * TD:2 * TD + 2] + sp['b1'][None]
    cb = jnp.pad(cb, ((0, NPAD - N), (0, 0)))
    ib = (intent @ w1[2 * TD + 2:])[:, None, :]
    q, k, v, s = _proj(vt, it, w1[:TD], w1[TD:2 * TD], cb, ib,
                       sp['w2'], sp['b2'].reshape(1, 1),
                       ip_['wq'], ip_['bq'][None], ip_['wk'], ip_['bk'][None],
                       ip_['wv'], ip_['bv'][None])
    srow = jnp.concatenate(
        [s[:, :N, 0], jnp.full((B, NPAD - N), -3e38, _f32)], axis=1)
    mrow, mcol = _topk(srow[:, None, :])
    tokens = _attn(q, k, v, mrow, mcol, vt, it, ip_['wo'], ip_['bo'][None])
    return _unpatchify(tokens[:, :N])


def kernel(vis_feat, ir_feat, params):
    kk = jnp.arange(H, dtype=_f32)
    ang = (-2.0 * jnp.pi / H) * (kk[:, None] * kk[None, :])
    scale = 1.0 / jnp.sqrt(float(H))
    fr = jnp.cos(ang) * scale
    fi = jnp.sin(ang) * scale

    x_all = jnp.concatenate([vis_feat, ir_feat], axis=0).reshape(2 * B * C, H, W)
    amp_all, ph_all, mean_all = _fft(x_all, fr, fi)
    va = amp_all[:B * C].reshape(B, C, H, W)
    ia = amp_all[B * C:].reshape(B, C, H, W)
    vp = ph_all[:B * C].reshape(B, C, H, W)
    ipp = ph_all[B * C:].reshape(B, C, H, W)
    gv = mean_all[:B * C].reshape(B, C)
    gi = mean_all[B * C:].reshape(B, C)

    logits = jnp.concatenate([gv, gi], axis=-1) @ params['router_w'] \
        + params['router_b']
    intent = jax.nn.softmax(logits, axis=-1) @ params['prompt_bank']

    ys, xs = jnp.meshgrid(jnp.arange(HP), jnp.arange(HP), indexing='ij')
    coords = jnp.stack([ys / (HP - 1), xs / (HP - 1)],
                       axis=-1).reshape(-1, 2).astype(_f32)

    fa = _branch(va, ia, intent, coords, params['amp_score'],
                 params['amp_inter'])
    fp = _branch(vp, ipp, intent, coords, params['phase_score'],
                 params['phase_inter'])

    spatial = _ifft(fa.reshape(B * C, H, W), fp.reshape(B * C, H, W), fr, fi)
    spatial = spatial.reshape(B, C, H, W)

    def w9(wname):
        return params[wname].transpose(2, 3, 0, 1).reshape(9, C, C)

    h1 = _conv(spatial, w9('conv1_w'), params['conv1_b'][:, None])
    return _conv(h1, w9('conv2_w'), params['conv2_b'][:, None],
                 vis_feat, ir_feat)
